# baseline (device time: 15179 ns/iter reference)
import jax
import jax.numpy as jnp
from jax import lax
from jax.experimental import pallas as pl
from jax.experimental.pallas import tpu as pltpu

N_DEV = 8
M = 512
N = 512
M_PER = M // N_DEV


def kernel(A, B):
    def body(a_ref, b_ref, out_ref, p_ref, comm_ref, send_sems, recv_sems):
        me = lax.axis_index("i")

        barrier = pltpu.get_barrier_semaphore()
        for d in range(1, N_DEV):
            pl.semaphore_signal(
                barrier,
                inc=1,
                device_id=((me + d) % N_DEV,),
                device_id_type=pl.DeviceIdType.MESH,
            )
        pl.semaphore_wait(barrier, N_DEV - 1)

        p_ref[...] = jnp.dot(
            a_ref[...].astype(jnp.bfloat16),
            b_ref[...].astype(jnp.bfloat16),
            preferred_element_type=jnp.float32,
        )

        rdmas = []
        for d in range(1, N_DEV):
            tgt = (me + d) % N_DEV
            rdma = pltpu.make_async_remote_copy(
                src_ref=p_ref.at[pl.ds(tgt * M_PER, M_PER), :],
                dst_ref=comm_ref.at[d],
                send_sem=send_sems.at[d],
                recv_sem=recv_sems.at[d],
                device_id=(tgt,),
                device_id_type=pl.DeviceIdType.MESH,
            )
            rdma.start()
            rdmas.append(rdma)

        acc = p_ref[pl.ds(me * M_PER, M_PER), :]
        for d in range(1, N_DEV):
            rdmas[d - 1].wait_recv()
            acc = acc + comm_ref[d]
        out_ref[...] = acc

        for d in range(1, N_DEV):
            rdmas[d - 1].wait_send()

    return pl.pallas_call(
        body,
        out_shape=jax.ShapeDtypeStruct((M_PER, N), jnp.float32),
        in_specs=[
            pl.BlockSpec(memory_space=pltpu.VMEM),
            pl.BlockSpec(memory_space=pltpu.VMEM),
        ],
        out_specs=pl.BlockSpec(memory_space=pltpu.VMEM),
        scratch_shapes=[
            pltpu.VMEM((M, N), jnp.float32),
            pltpu.VMEM((N_DEV, M_PER, N), jnp.float32),
            pltpu.SemaphoreType.DMA((N_DEV,)),
            pltpu.SemaphoreType.DMA((N_DEV,)),
        ],
        compiler_params=pltpu.CompilerParams(collective_id=0),
    )(A, B)


# device time: 12313 ns/iter; 1.2328x vs baseline; 1.2328x over previous
import jax
import jax.numpy as jnp
from jax import lax
from jax.experimental import pallas as pl
from jax.experimental.pallas import tpu as pltpu

N_DEV = 8
M = 512
N = 512
M_PER = M // N_DEV


def kernel(A, B):
    def body(a_ref, b_ref, out_ref, p_ref, comm_ref, send_sems, recv_sems):
        me = lax.axis_index("i")

        barrier = pltpu.get_barrier_semaphore()
        for d in range(1, N_DEV):
            pl.semaphore_signal(
                barrier,
                inc=1,
                device_id=((me + d) % N_DEV,),
                device_id_type=pl.DeviceIdType.MESH,
            )
        pl.semaphore_wait(barrier, N_DEV - 1)

        p_ref[...] = jnp.dot(
            a_ref[...].astype(jnp.bfloat16),
            b_ref[...].astype(jnp.bfloat16),
            preferred_element_type=jnp.float32,
        ).astype(jnp.bfloat16)

        rdmas = []
        for d in range(1, N_DEV):
            tgt = (me + d) % N_DEV
            rdma = pltpu.make_async_remote_copy(
                src_ref=p_ref.at[pl.ds(tgt * M_PER, M_PER), :],
                dst_ref=comm_ref.at[d],
                send_sem=send_sems.at[d],
                recv_sem=recv_sems.at[d],
                device_id=(tgt,),
                device_id_type=pl.DeviceIdType.MESH,
            )
            rdma.start()
            rdmas.append(rdma)

        acc = p_ref[pl.ds(me * M_PER, M_PER), :].astype(jnp.float32)
        for d in range(1, N_DEV):
            rdmas[d - 1].wait_recv()
            acc = acc + comm_ref[d].astype(jnp.float32)
        out_ref[...] = acc

        for d in range(1, N_DEV):
            rdmas[d - 1].wait_send()

    return pl.pallas_call(
        body,
        out_shape=jax.ShapeDtypeStruct((M_PER, N), jnp.float32),
        in_specs=[
            pl.BlockSpec(memory_space=pltpu.VMEM),
            pl.BlockSpec(memory_space=pltpu.VMEM),
        ],
        out_specs=pl.BlockSpec(memory_space=pltpu.VMEM),
        scratch_shapes=[
            pltpu.VMEM((M, N), jnp.bfloat16),
            pltpu.VMEM((N_DEV, M_PER, N), jnp.bfloat16),
            pltpu.SemaphoreType.DMA((N_DEV,)),
            pltpu.SemaphoreType.DMA((N_DEV,)),
        ],
        compiler_params=pltpu.CompilerParams(collective_id=0),
    )(A, B)


# device time: 7653 ns/iter; 1.9834x vs baseline; 1.6089x over previous
import jax
import jax.numpy as jnp
from jax import lax
from jax.experimental import pallas as pl
from jax.experimental.pallas import tpu as pltpu

N_DEV = 8
M = 512
N = 512
M_PER = M // N_DEV


def kernel(A, B):
    def body(a_ref, b_ref, out_ref, p_ref, comm_ref, send_sems, recv_sems):
        me = lax.axis_index("i")

        barrier = pltpu.get_barrier_semaphore()
        for d in range(1, N_DEV):
            pl.semaphore_signal(
                barrier,
                inc=1,
                device_id=((me + d) % N_DEV,),
                device_id_type=pl.DeviceIdType.MESH,
            )

        b16 = b_ref[...].astype(jnp.bfloat16)

        rdmas = []
        for d in range(1, N_DEV):
            tgt = (me + d) % N_DEV
            rows = pl.ds(tgt * M_PER, M_PER)
            p_ref[rows, :] = jnp.dot(
                a_ref[rows, :].astype(jnp.bfloat16),
                b16,
                preferred_element_type=jnp.float32,
            ).astype(jnp.bfloat16)
            if d == 1:
                pl.semaphore_wait(barrier, N_DEV - 1)
            rdma = pltpu.make_async_remote_copy(
                src_ref=p_ref.at[rows, :],
                dst_ref=comm_ref.at[d],
                send_sem=send_sems.at[d],
                recv_sem=recv_sems.at[d],
                device_id=(tgt,),
                device_id_type=pl.DeviceIdType.MESH,
            )
            rdma.start()
            rdmas.append(rdma)

        own_rows = pl.ds(me * M_PER, M_PER)
        p_ref[own_rows, :] = jnp.dot(
            a_ref[own_rows, :].astype(jnp.bfloat16),
            b16,
            preferred_element_type=jnp.float32,
        ).astype(jnp.bfloat16)

        acc = p_ref[pl.ds(me * M_PER, M_PER), :].astype(jnp.float32)
        for d in range(1, N_DEV):
            rdmas[d - 1].wait_recv()
            acc = acc + comm_ref[d].astype(jnp.float32)
        out_ref[...] = acc

        for d in range(1, N_DEV):
            rdmas[d - 1].wait_send()

    return pl.pallas_call(
        body,
        out_shape=jax.ShapeDtypeStruct((M_PER, N), jnp.float32),
        in_specs=[
            pl.BlockSpec(memory_space=pltpu.VMEM),
            pl.BlockSpec(memory_space=pltpu.VMEM),
        ],
        out_specs=pl.BlockSpec(memory_space=pltpu.VMEM),
        scratch_shapes=[
            pltpu.VMEM((M, N), jnp.bfloat16),
            pltpu.VMEM((N_DEV, M_PER, N), jnp.bfloat16),
            pltpu.SemaphoreType.DMA((N_DEV,)),
            pltpu.SemaphoreType.DMA((N_DEV,)),
        ],
        compiler_params=pltpu.CompilerParams(collective_id=0),
    )(A, B)
